# Initial kernel scaffold; baseline (speedup 1.0000x reference)
import jax
import jax.numpy as jnp
from jax.experimental import pallas as pl


def kernel(x, edge_index, W_l0, b_l0, W_r0, W_l1, b_l1, W_r1, W_lin0, b_lin0, W_out, b_out):
    raise NotImplementedError("write your pallas kernel here")


# speedup vs baseline: 2.4342x; 2.4342x over previous
import functools

import jax
import jax.numpy as jnp
from jax import lax
from jax.experimental import pallas as pl
from jax.experimental.pallas import tpu as pltpu
from jax.experimental.pallas import tpu_sc as plsc

N = 10000
NPAD = 10240
E = 320000
D = 128
OUT = 16

NC = 2
NS = 16
NW = NC * NS
CH = 80
NCHUNK = 128
EPAD = NW * NCHUNK * CH
EPW = NCHUNK * CH
TPT = NPAD // NS
WCH = TPT // CH
CW = 16
PG = 16

_HI = lax.Precision.HIGHEST


def _sc_segment_sum(src_hbm, dst_hbm, x_hbm, ridx_hbm, zrow_hbm,
                    outp_hbm, src_w, dst_w, zidx_w, rows_v, acc_sh, sem):
    c = lax.axis_index("c")
    s = lax.axis_index("s")
    wid = s * NC + c
    tbase = s * TPT

    pltpu.sync_copy(zrow_hbm, rows_v)
    pltpu.sync_copy(ridx_hbm.at[s], zidx_w)

    for k in range(WCH):
        pltpu.sync_copy(rows_v, acc_sh.at[zidx_w.at[k]])
    plsc.subcore_barrier()

    def body(j, carry):
        pltpu.async_copy(x_hbm.at[src_w.at[j]], rows_v, sem).wait()
        pltpu.sync_copy(rows_v, acc_sh.at[dst_w.at[j]], add=True)
        return carry

    for pg in range(NCHUNK // PG):
        pltpu.sync_copy(src_hbm.at[wid, pl.ds(pg * PG, PG)], src_w)
        pltpu.sync_copy(dst_hbm.at[wid, pl.ds(pg * PG, PG)], dst_w)
        lax.fori_loop(0, PG, body, 0)
    plsc.subcore_barrier()

    obase = c * NPAD + tbase
    for k in range(WCH):
        pltpu.sync_copy(acc_sh.at[zidx_w.at[k]], rows_v)
        pltpu.sync_copy(rows_v, outp_hbm.at[pl.ds(obase + k * CH, CH)])


def _sc_counts(dst_hbm, ridx_hbm, zrow_hbm, ones_hbm, outc_hbm,
               dst_w, zidx_w, zrow_v, ones_v, cnt_sh):
    c = lax.axis_index("c")
    s = lax.axis_index("s")
    wid = s * NC + c
    tbase = s * TPT

    pltpu.sync_copy(zrow_hbm, zrow_v)
    pltpu.sync_copy(ones_hbm, ones_v)
    pltpu.sync_copy(ridx_hbm.at[s], zidx_w)

    for k in range(WCH):
        pltpu.sync_copy(zrow_v, cnt_sh.at[zidx_w.at[k]])
    plsc.subcore_barrier()

    def body(j, carry):
        pltpu.sync_copy(ones_v, cnt_sh.at[dst_w.at[j]], add=True)
        return carry

    for pg in range(NCHUNK // PG):
        pltpu.sync_copy(dst_hbm.at[wid, pl.ds(pg * PG, PG)], dst_w)
        lax.fori_loop(0, PG, body, 0)
    plsc.subcore_barrier()

    obase = c * NPAD + tbase
    for k in range(WCH):
        pltpu.sync_copy(cnt_sh.at[zidx_w.at[k]], zrow_v)
        pltpu.sync_copy(zrow_v, outc_hbm.at[pl.ds(obase + k * CH, CH)])


def _make_sc_call():
    mesh = plsc.VectorSubcoreMesh(core_axis_name="c", subcore_axis_name="s")
    return pl.kernel(
        _sc_segment_sum,
        out_type=(jax.ShapeDtypeStruct((NC * NPAD, D), jnp.float32),),
        mesh=mesh,
        scratch_types=[
            pltpu.VMEM((PG, CH), jnp.int32),
            pltpu.VMEM((PG, CH), jnp.int32),
            pltpu.VMEM((WCH, CH), jnp.int32),
            pltpu.VMEM((CH, D), jnp.float32),
            pltpu.VMEM_SHARED((NPAD, D), jnp.float32),
            pltpu.SemaphoreType.DMA,
        ],
    )


def _make_cnt_call():
    mesh = plsc.VectorSubcoreMesh(core_axis_name="c", subcore_axis_name="s")
    return pl.kernel(
        _sc_counts,
        out_type=(jax.ShapeDtypeStruct((NC * NPAD, D), jnp.float32),),
        mesh=mesh,
        scratch_types=[
            pltpu.VMEM((PG, CH), jnp.int32),
            pltpu.VMEM((WCH, CH), jnp.int32),
            pltpu.VMEM((CH, D), jnp.float32),
            pltpu.VMEM((CH, D), jnp.float32),
            pltpu.VMEM_SHARED((NPAD, D), jnp.float32),
        ],
    )


def _tc_layer0(p0, p1, c0, c1, x_blk, Wl, bl, Wr, h_ref):
    cnt = c0[:, 0:1] + c1[:, 0:1]
    inv = 1.0 / jnp.maximum(cnt, 1.0)
    agg = (p0[...] + p1[...]) * inv
    h = lax.dot_general(agg, Wl[...], (((1,), (1,)), ((), ())),
                        precision=_HI, preferred_element_type=jnp.float32)
    h = h + lax.dot_general(x_blk[...], Wr[...], (((1,), (1,)), ((), ())),
                            precision=_HI, preferred_element_type=jnp.float32)
    h_ref[...] = jnp.maximum(h + bl[...], 0.0)


def _tc_layer1(p0, p1, c0, c1, h_blk, Wl, bl, Wr, Wlin, blin, Wout, bout,
               out_ref, acc):
    cnt = c0[:, 0:1] + c1[:, 0:1]
    inv = 1.0 / jnp.maximum(cnt, 1.0)
    agg = (p0[...] + p1[...]) * inv
    h2 = lax.dot_general(agg, Wl[...], (((1,), (1,)), ((), ())),
                         precision=_HI, preferred_element_type=jnp.float32)
    h2 = h2 + lax.dot_general(h_blk[...], Wr[...], (((1,), (1,)), ((), ())),
                              precision=_HI, preferred_element_type=jnp.float32)
    h2 = jnp.maximum(h2 + bl[...], 0.0)

    i = pl.program_id(0)
    row = i * _BM + lax.broadcasted_iota(jnp.int32, (_BM, 1), 0)
    h2 = jnp.where(row < N, h2, 0.0)
    part = jnp.sum(h2, axis=0, keepdims=True)

    @pl.when(i == 0)
    def _():
        acc[0:1, :] = part

    @pl.when(i > 0)
    def _():
        acc[0:1, :] = acc[0:1, :] + part

    @pl.when(i == pl.num_programs(0) - 1)
    def _():
        g = acc[0:1, :] * (1.0 / N)
        g = lax.dot_general(g, Wlin[...], (((1,), (1,)), ((), ())),
                            precision=_HI, preferred_element_type=jnp.float32)
        g = jnp.maximum(g + blin[...], 0.0)
        o = lax.dot_general(g, Wout[...], (((1,), (1,)), ((), ())),
                            precision=_HI, preferred_element_type=jnp.float32)
        out_ref[...] = o + bout[...]


_BM = 2048
_NBLK = NPAD // _BM


def _row_spec(i_off=0):
    return pl.BlockSpec((_BM, D), lambda i, o=i_off: (i + o, 0))


def _cnt_spec(i_off=0):
    return pl.BlockSpec((_BM, D), lambda i, o=i_off: (i + o, 0))


def _full_spec(r, c):
    return pl.BlockSpec((r, c), lambda i: (0, 0))


def _layer0_call(pp, cc, x, Wl, bl, Wr):
    return pl.pallas_call(
        _tc_layer0,
        grid=(_NBLK,),
        in_specs=[
            _row_spec(0), _row_spec(_NBLK), _cnt_spec(0), _cnt_spec(_NBLK),
            _row_spec(0), _full_spec(D, D), _full_spec(1, D), _full_spec(D, D),
        ],
        out_specs=_row_spec(0),
        out_shape=jax.ShapeDtypeStruct((NPAD, D), jnp.float32),
    )(pp, pp, cc, cc, x, Wl, bl, Wr)


def _layer1_call(pp, cc, h, Wl, bl, Wr, Wlin, blin, Wout, bout):
    return pl.pallas_call(
        _tc_layer1,
        grid=(_NBLK,),
        in_specs=[
            _row_spec(0), _row_spec(_NBLK), _cnt_spec(0), _cnt_spec(_NBLK),
            _row_spec(0), _full_spec(D, D), _full_spec(1, D), _full_spec(D, D),
            _full_spec(D, D), _full_spec(1, D), _full_spec(OUT, D),
            _full_spec(1, OUT),
        ],
        out_specs=pl.BlockSpec((1, OUT), lambda i: (0, 0)),
        out_shape=jax.ShapeDtypeStruct((1, OUT), jnp.float32),
        scratch_shapes=[pltpu.VMEM((8, D), jnp.float32)],
    )(pp, pp, cc, cc, h, Wl, bl, Wr, Wlin, blin, Wout, bout)


def kernel(x, edge_index, W_l0, b_l0, W_r0, W_l1, b_l1, W_r1,
           W_lin0, b_lin0, W_out, b_out):
    pad_src = jnp.zeros((EPAD - E,), jnp.int32)
    pad_dst = jnp.full((EPAD - E,), NPAD - 1, jnp.int32)
    src3 = jnp.concatenate([edge_index[0], pad_src]).reshape(NW, NCHUNK, CH)
    dst3 = jnp.concatenate([edge_index[1], pad_dst]).reshape(NW, NCHUNK, CH)
    x_pad = jnp.pad(x, ((0, NPAD - N), (0, 0)))
    ridx = jnp.arange(NPAD, dtype=jnp.int32).reshape(NS, WCH, CH)
    zrow = jnp.zeros((CH, D), jnp.float32)
    onesD = jnp.ones((CH, D), jnp.float32)

    sc = _make_sc_call()
    cntk = _make_cnt_call()

    (cc,) = cntk(dst3, ridx, zrow, onesD)
    (pp0,) = sc(src3, dst3, x_pad, ridx, zrow)
    h = _layer0_call(pp0, cc, x_pad, W_l0, b_l0.reshape(1, D), W_r0)
    (pp1,) = sc(src3, dst3, h, ridx, zrow)
    out = _layer1_call(pp1, cc, h, W_l1, b_l1.reshape(1, D), W_r1,
                       W_lin0, b_lin0.reshape(1, D), W_out,
                       b_out.reshape(1, OUT))
    return out


# speedup vs baseline: 2.7662x; 1.1364x over previous
import functools

import jax
import jax.numpy as jnp
from jax import lax
from jax.experimental import pallas as pl
from jax.experimental.pallas import tpu as pltpu
from jax.experimental.pallas import tpu_sc as plsc

N = 10000
NPAD = 10240
E = 320000
D = 128
OUT = 16

NC = 2
NS = 16
NW = NC * NS
CH = 80
NCHUNK = 128
EPAD = NW * NCHUNK * CH
EPW = NCHUNK * CH
TPT = NPAD // NS
WCH = TPT // CH
CW = 16
PG = 16

_HI = lax.Precision.HIGHEST


def _sc_segment_sum(src_hbm, dst_hbm, x_hbm, ridx_hbm, zrow_hbm,
                    outp_hbm, src_w, dst_w, zidx_w, rows_v, rows2_v,
                    acc_sh, sem, sem2):
    c = lax.axis_index("c")
    s = lax.axis_index("s")
    wid = s * NC + c
    tbase = s * TPT

    pltpu.sync_copy(zrow_hbm, rows_v)
    pltpu.sync_copy(ridx_hbm.at[s], zidx_w)

    for k in range(WCH):
        pltpu.sync_copy(rows_v, acc_sh.at[zidx_w.at[k]])
    plsc.subcore_barrier()

    bufs = (rows_v, rows2_v)
    sems = (sem, sem2)
    for pg in range(NCHUNK // PG):
        pltpu.sync_copy(src_hbm.at[wid, pl.ds(pg * PG, PG)], src_w)
        pltpu.sync_copy(dst_hbm.at[wid, pl.ds(pg * PG, PG)], dst_w)
        pltpu.async_copy(x_hbm.at[src_w.at[0]], bufs[0], sems[0])
        for j in range(PG):
            cur, nxt = bufs[j % 2], bufs[(j + 1) % 2]
            if j + 1 < PG:
                pltpu.async_copy(x_hbm.at[src_w.at[j + 1]], nxt,
                                 sems[(j + 1) % 2])
            pltpu.make_async_copy(x_hbm.at[src_w.at[j]], cur,
                                  sems[j % 2]).wait()
            pltpu.sync_copy(cur, acc_sh.at[dst_w.at[j]], add=True)
    plsc.subcore_barrier()

    obase = c * NPAD + tbase
    for k in range(WCH):
        pltpu.sync_copy(acc_sh.at[zidx_w.at[k]], rows_v)
        pltpu.sync_copy(rows_v, outp_hbm.at[pl.ds(obase + k * CH, CH)])


def _sc_counts(dst_hbm, ridx_hbm, zrow_hbm, ones_hbm, outc_hbm,
               dst_w, zidx_w, zrow_v, ones_v, cnt_sh):
    c = lax.axis_index("c")
    s = lax.axis_index("s")
    wid = s * NC + c
    tbase = s * TPT

    pltpu.sync_copy(zrow_hbm, zrow_v)
    pltpu.sync_copy(ones_hbm, ones_v)
    pltpu.sync_copy(ridx_hbm.at[s], zidx_w)

    for k in range(WCH):
        pltpu.sync_copy(zrow_v, cnt_sh.at[zidx_w.at[k]])
    plsc.subcore_barrier()

    def body(j, carry):
        pltpu.sync_copy(ones_v, cnt_sh.at[dst_w.at[j]], add=True)
        return carry

    for pg in range(NCHUNK // PG):
        pltpu.sync_copy(dst_hbm.at[wid, pl.ds(pg * PG, PG)], dst_w)
        lax.fori_loop(0, PG, body, 0)
    plsc.subcore_barrier()

    obase = c * NPAD + tbase
    for k in range(WCH):
        pltpu.sync_copy(cnt_sh.at[zidx_w.at[k]], zrow_v)
        pltpu.sync_copy(zrow_v, outc_hbm.at[pl.ds(obase + k * CH, CH)])


def _make_sc_call():
    mesh = plsc.VectorSubcoreMesh(core_axis_name="c", subcore_axis_name="s")
    return pl.kernel(
        _sc_segment_sum,
        out_type=(jax.ShapeDtypeStruct((NC * NPAD, D), jnp.float32),),
        mesh=mesh,
        scratch_types=[
            pltpu.VMEM((PG, CH), jnp.int32),
            pltpu.VMEM((PG, CH), jnp.int32),
            pltpu.VMEM((WCH, CH), jnp.int32),
            pltpu.VMEM((CH, D), jnp.float32),
            pltpu.VMEM((CH, D), jnp.float32),
            pltpu.VMEM_SHARED((NPAD, D), jnp.float32),
            pltpu.SemaphoreType.DMA,
            pltpu.SemaphoreType.DMA,
        ],
    )


def _make_cnt_call():
    mesh = plsc.VectorSubcoreMesh(core_axis_name="c", subcore_axis_name="s")
    return pl.kernel(
        _sc_counts,
        out_type=(jax.ShapeDtypeStruct((NC * NPAD, D), jnp.float32),),
        mesh=mesh,
        scratch_types=[
            pltpu.VMEM((PG, CH), jnp.int32),
            pltpu.VMEM((WCH, CH), jnp.int32),
            pltpu.VMEM((CH, D), jnp.float32),
            pltpu.VMEM((CH, D), jnp.float32),
            pltpu.VMEM_SHARED((NPAD, D), jnp.float32),
        ],
    )


def _tc_layer0(p0, p1, c0, c1, x_blk, Wl, bl, Wr, h_ref):
    cnt = c0[:, 0:1] + c1[:, 0:1]
    inv = 1.0 / jnp.maximum(cnt, 1.0)
    agg = (p0[...] + p1[...]) * inv
    h = lax.dot_general(agg, Wl[...], (((1,), (1,)), ((), ())),
                        precision=_HI, preferred_element_type=jnp.float32)
    h = h + lax.dot_general(x_blk[...], Wr[...], (((1,), (1,)), ((), ())),
                            precision=_HI, preferred_element_type=jnp.float32)
    h_ref[...] = jnp.maximum(h + bl[...], 0.0)


def _tc_layer1(p0, p1, c0, c1, h_blk, Wl, bl, Wr, Wlin, blin, Wout, bout,
               out_ref, acc):
    cnt = c0[:, 0:1] + c1[:, 0:1]
    inv = 1.0 / jnp.maximum(cnt, 1.0)
    agg = (p0[...] + p1[...]) * inv
    h2 = lax.dot_general(agg, Wl[...], (((1,), (1,)), ((), ())),
                         precision=_HI, preferred_element_type=jnp.float32)
    h2 = h2 + lax.dot_general(h_blk[...], Wr[...], (((1,), (1,)), ((), ())),
                              precision=_HI, preferred_element_type=jnp.float32)
    h2 = jnp.maximum(h2 + bl[...], 0.0)

    i = pl.program_id(0)
    row = i * _BM + lax.broadcasted_iota(jnp.int32, (_BM, 1), 0)
    h2 = jnp.where(row < N, h2, 0.0)
    part = jnp.sum(h2, axis=0, keepdims=True)

    @pl.when(i == 0)
    def _():
        acc[0:1, :] = part

    @pl.when(i > 0)
    def _():
        acc[0:1, :] = acc[0:1, :] + part

    @pl.when(i == pl.num_programs(0) - 1)
    def _():
        g = acc[0:1, :] * (1.0 / N)
        g = lax.dot_general(g, Wlin[...], (((1,), (1,)), ((), ())),
                            precision=_HI, preferred_element_type=jnp.float32)
        g = jnp.maximum(g + blin[...], 0.0)
        o = lax.dot_general(g, Wout[...], (((1,), (1,)), ((), ())),
                            precision=_HI, preferred_element_type=jnp.float32)
        out_ref[...] = o + bout[...]


_BM = 2048
_NBLK = NPAD // _BM


def _row_spec(i_off=0):
    return pl.BlockSpec((_BM, D), lambda i, o=i_off: (i + o, 0))


def _cnt_spec(i_off=0):
    return pl.BlockSpec((_BM, D), lambda i, o=i_off: (i + o, 0))


def _full_spec(r, c):
    return pl.BlockSpec((r, c), lambda i: (0, 0))


def _layer0_call(pp, cc, x, Wl, bl, Wr):
    return pl.pallas_call(
        _tc_layer0,
        grid=(_NBLK,),
        in_specs=[
            _row_spec(0), _row_spec(_NBLK), _cnt_spec(0), _cnt_spec(_NBLK),
            _row_spec(0), _full_spec(D, D), _full_spec(1, D), _full_spec(D, D),
        ],
        out_specs=_row_spec(0),
        out_shape=jax.ShapeDtypeStruct((NPAD, D), jnp.float32),
    )(pp, pp, cc, cc, x, Wl, bl, Wr)


def _layer1_call(pp, cc, h, Wl, bl, Wr, Wlin, blin, Wout, bout):
    return pl.pallas_call(
        _tc_layer1,
        grid=(_NBLK,),
        in_specs=[
            _row_spec(0), _row_spec(_NBLK), _cnt_spec(0), _cnt_spec(_NBLK),
            _row_spec(0), _full_spec(D, D), _full_spec(1, D), _full_spec(D, D),
            _full_spec(D, D), _full_spec(1, D), _full_spec(OUT, D),
            _full_spec(1, OUT),
        ],
        out_specs=pl.BlockSpec((1, OUT), lambda i: (0, 0)),
        out_shape=jax.ShapeDtypeStruct((1, OUT), jnp.float32),
        scratch_shapes=[pltpu.VMEM((8, D), jnp.float32)],
    )(pp, pp, cc, cc, h, Wl, bl, Wr, Wlin, blin, Wout, bout)


def kernel(x, edge_index, W_l0, b_l0, W_r0, W_l1, b_l1, W_r1,
           W_lin0, b_lin0, W_out, b_out):
    pad_src = jnp.zeros((EPAD - E,), jnp.int32)
    pad_dst = jnp.full((EPAD - E,), NPAD - 1, jnp.int32)
    src3 = jnp.concatenate([edge_index[0], pad_src]).reshape(NW, NCHUNK, CH)
    dst3 = jnp.concatenate([edge_index[1], pad_dst]).reshape(NW, NCHUNK, CH)
    x_pad = jnp.pad(x, ((0, NPAD - N), (0, 0)))
    ridx = jnp.arange(NPAD, dtype=jnp.int32).reshape(NS, WCH, CH)
    zrow = jnp.zeros((CH, D), jnp.float32)
    onesD = jnp.ones((CH, D), jnp.float32)

    sc = _make_sc_call()
    cntk = _make_cnt_call()

    (cc,) = cntk(dst3, ridx, zrow, onesD)
    (pp0,) = sc(src3, dst3, x_pad, ridx, zrow)
    h = _layer0_call(pp0, cc, x_pad, W_l0, b_l0.reshape(1, D), W_r0)
    (pp1,) = sc(src3, dst3, h, ridx, zrow)
    out = _layer1_call(pp1, cc, h, W_l1, b_l1.reshape(1, D), W_r1,
                       W_lin0, b_lin0.reshape(1, D), W_out,
                       b_out.reshape(1, OUT))
    return out
